# Initial kernel scaffold; baseline (speedup 1.0000x reference)
import jax
import jax.numpy as jnp
from jax.experimental import pallas as pl


def kernel(x, position, level_weight):
    raise NotImplementedError("write your pallas kernel here")


# speedup vs baseline: 3.9204x; 3.9204x over previous
import functools

import jax
import jax.numpy as jnp
from jax.experimental import pallas as pl
from jax.experimental.pallas import tpu as pltpu

_LEVELS = 256
_DBLK = 2048


def _body(x_ref, pos_ref, lw_ref, o_ref, *, P, levels):
    xr = x_ref[0, 0, :]
    idx = jnp.clip(jnp.round(xr * (levels - 1)), 0, levels - 1).astype(jnp.int32)
    ii = jax.lax.broadcasted_iota(jnp.int32, (levels, P), 0)
    at = (ii == idx[None, :]).astype(jnp.bfloat16)
    g = jnp.dot(at, pos_ref[...], preferred_element_type=jnp.float32)
    s = jnp.sum(g * lw_ref[...], axis=0)
    o_ref[0, 0, :] = jnp.where(s > 0, 1.0, -1.0).astype(jnp.float32)


@jax.jit
def kernel(x, position, level_weight):
    B = x.shape[0]
    P = x.shape[1] * x.shape[2]
    D = position.shape[1]
    levels = level_weight.shape[0]
    x_flat = x.reshape(B, 1, P)
    pos_bf = position.astype(jnp.bfloat16)

    dblk = min(_DBLK, D)
    grid = (D // dblk, B)
    out = pl.pallas_call(
        functools.partial(_body, P=P, levels=levels),
        grid=grid,
        in_specs=[
            pl.BlockSpec((1, 1, P), lambda j, b: (b, 0, 0)),
            pl.BlockSpec((P, dblk), lambda j, b: (0, j)),
            pl.BlockSpec((levels, dblk), lambda j, b: (0, j)),
        ],
        out_specs=pl.BlockSpec((1, 1, dblk), lambda j, b: (b, 0, j)),
        out_shape=jax.ShapeDtypeStruct((B, 1, D), jnp.float32),
        compiler_params=pltpu.CompilerParams(
            dimension_semantics=("arbitrary", "arbitrary"),
        ),
    )(x_flat, pos_bf, level_weight)
    return out.reshape(B, D)


# speedup vs baseline: 4.2904x; 1.0944x over previous
import functools

import jax
import jax.numpy as jnp
from jax.experimental import pallas as pl
from jax.experimental.pallas import tpu as pltpu

_LEVELS = 256
_DBLK = 4096


def _body(x_ref, pos_ref, lw_ref, o_ref, *, P, levels):
    xr = x_ref[0, 0, :]
    idx = jnp.clip(jnp.round(xr * (levels - 1)), 0, levels - 1).astype(jnp.int32)
    ii = jax.lax.broadcasted_iota(jnp.int32, (levels, P), 0)
    at = (ii == idx[None, :]).astype(jnp.bfloat16)
    g = jnp.dot(at, pos_ref[...], preferred_element_type=jnp.float32)
    s = jnp.sum(g * lw_ref[...], axis=0)
    o_ref[0, 0, :] = jnp.where(s > 0, 1.0, -1.0).astype(jnp.float32)


@jax.jit
def kernel(x, position, level_weight):
    B = x.shape[0]
    P = x.shape[1] * x.shape[2]
    D = position.shape[1]
    levels = level_weight.shape[0]
    x_flat = x.reshape(B, 1, P)
    pos_bf = position.astype(jnp.bfloat16)

    dblk = min(_DBLK, D)
    grid = (D // dblk, B)
    out = pl.pallas_call(
        functools.partial(_body, P=P, levels=levels),
        grid=grid,
        in_specs=[
            pl.BlockSpec((1, 1, P), lambda j, b: (b, 0, 0)),
            pl.BlockSpec((P, dblk), lambda j, b: (0, j)),
            pl.BlockSpec((levels, dblk), lambda j, b: (0, j)),
        ],
        out_specs=pl.BlockSpec((1, 1, dblk), lambda j, b: (b, 0, j)),
        out_shape=jax.ShapeDtypeStruct((B, 1, D), jnp.float32),
        compiler_params=pltpu.CompilerParams(
            dimension_semantics=("arbitrary", "arbitrary"),
        ),
    )(x_flat, pos_bf, level_weight)
    return out.reshape(B, D)
